# Initial kernel scaffold; baseline (speedup 1.0000x reference)
import jax
import jax.numpy as jnp
from jax.experimental import pallas as pl


def kernel(x, emb, W1, b1, W2, b2, Wout, bout):
    raise NotImplementedError("write your pallas kernel here")


# speedup vs baseline: 1.0184x; 1.0184x over previous
import functools

import jax
import jax.numpy as jnp
import numpy as np
from jax import lax
from jax.experimental import pallas as pl
from jax.experimental.pallas import tpu as pltpu
from jax.experimental.pallas import tpu_sc as plsc

N_DENSE = 13
N_SPARSE = 26
K = 16
VOCAB = 100000
B = 4096
D_SPARSE = N_SPARSE * K
D = D_SPARSE + N_DENSE
DP = 512
DENSE_PAD = DP - D_SPARSE

NW = 32
LOOKUPS = B * N_SPARSE
PER_W = LOOKUPS // NW
CHUNK = 128
NCH = PER_W // CHUNK


def _sc_gather(table, idx3):
    mesh = plsc.VectorSubcoreMesh(core_axis_name="c", subcore_axis_name="s")

    @functools.partial(
        pl.kernel,
        mesh=mesh,
        out_type=jax.ShapeDtypeStruct((LOOKUPS, K), jnp.float32),
        compiler_params=pltpu.CompilerParams(use_tc_tiling_on_sc=False),
        scratch_types=[
            pltpu.VMEM((NCH, CHUNK), jnp.int32),
            pltpu.VMEM((PER_W, K), jnp.float32),
            pltpu.SemaphoreType.DMA,
        ],
    )
    def k(table_hbm, idx_hbm, out_hbm, idx_v, rows_v, sem):
        wid = lax.axis_index("s") * 2 + lax.axis_index("c")
        pltpu.sync_copy(idx_hbm.at[wid], idx_v)
        handles = [
            pltpu.async_copy(
                table_hbm.at[idx_v.at[j]],
                rows_v.at[pl.ds(j * CHUNK, CHUNK)],
                sem,
            )
            for j in range(NCH)
        ]
        for h in handles:
            h.wait()
        pltpu.sync_copy(rows_v, out_hbm.at[pl.ds(wid * PER_W, PER_W)])

    return k(table, idx3)


def _mlp_body(se_ref, dp_ref, w1, b1r, w2, b2r, wo, bor, out_ref):
    dn = (((1,), (1,)), ((), ()))
    h = jnp.concatenate([se_ref[...], dp_ref[...]], axis=1)
    o = lax.dot_general(h, w1[...], dn,
                        preferred_element_type=jnp.float32) + b1r[...]
    h = jnp.maximum(h + o, 0.0)
    o = lax.dot_general(h, w2[...], dn,
                        preferred_element_type=jnp.float32) + b2r[...]
    h = jnp.maximum(h + o, 0.0)
    hb = h.astype(jnp.bfloat16).astype(jnp.float32)
    wob = wo[...].astype(jnp.bfloat16).astype(jnp.float32)
    s = jnp.sum(hb * wob, axis=1, keepdims=True) + bor[...]
    out_ref[...] = 1.0 / (1.0 + jnp.exp(-s))


def _tc_mlp(se, dp, w1p, b1p, w2p, b2p, wop, bop):
    bb = 1024
    return pl.pallas_call(
        _mlp_body,
        grid=(B // bb,),
        in_specs=[
            pl.BlockSpec((bb, D_SPARSE), lambda i: (i, 0)),
            pl.BlockSpec((bb, DENSE_PAD), lambda i: (i, 0)),
            pl.BlockSpec((DP, DP), lambda i: (0, 0)),
            pl.BlockSpec((1, DP), lambda i: (0, 0)),
            pl.BlockSpec((DP, DP), lambda i: (0, 0)),
            pl.BlockSpec((1, DP), lambda i: (0, 0)),
            pl.BlockSpec((1, DP), lambda i: (0, 0)),
            pl.BlockSpec((1, 1), lambda i: (0, 0)),
        ],
        out_specs=pl.BlockSpec((bb, 1), lambda i: (i, 0)),
        out_shape=jax.ShapeDtypeStruct((B, 1), jnp.float32),
    )(se, dp, w1p, b1p, w2p, b2p, wop, bop)


_PERM = np.concatenate([np.arange(N_DENSE, D), np.arange(N_DENSE)])


def kernel(x, emb, W1, b1, W2, b2, Wout, bout):
    x_dense = x[:, :N_DENSE]
    x_sparse = x[:, N_DENSE:].astype(jnp.int32)
    offs = jnp.arange(N_SPARSE, dtype=jnp.int32) * VOCAB
    idx3 = (x_sparse + offs[None, :]).reshape(NW, NCH, CHUNK)
    table = emb.reshape(N_SPARSE * VOCAB, K)

    se = _sc_gather(table, idx3).reshape(B, D_SPARSE)
    dp = jnp.pad(x_dense, ((0, 0), (0, DENSE_PAD - N_DENSE)))

    def pw(w):
        return jnp.pad(w[_PERM][:, _PERM], ((0, DP - D), (0, DP - D)))

    w1p, w2p = pw(W1), pw(W2)
    b1p = jnp.pad(b1[_PERM], (0, DP - D)).reshape(1, DP)
    b2p = jnp.pad(b2[_PERM], (0, DP - D)).reshape(1, DP)
    wop = jnp.pad(Wout[:, _PERM], ((0, 0), (0, DP - D)))
    bop = bout.reshape(1, 1)

    return _tc_mlp(se, dp, w1p, b1p, w2p, b2p, wop, bop)


# speedup vs baseline: 2.3132x; 2.2715x over previous
import functools

import jax
import jax.numpy as jnp
import numpy as np
from jax import lax
from jax.experimental import pallas as pl
from jax.experimental.pallas import tpu as pltpu
from jax.experimental.pallas import tpu_sc as plsc

N_DENSE = 13
N_SPARSE = 26
K = 16
VOCAB = 100000
B = 4096
D_SPARSE = N_SPARSE * K
D = D_SPARSE + N_DENSE
DP = 512
DENSE_PAD = DP - D_SPARSE

NW = 32

VMAIN = VOCAB - VOCAB % 128
TAIL = VOCAB - VMAIN
RWIN = 3584
RNW = 28
RLAST = VMAIN - 27 * RWIN
RTASK = N_SPARSE * RNW
RTPW = (RTASK + NW - 1) // NW
VSTRIDE = 100032
FROWS = VSTRIDE // 8
OUTA = N_SPARSE * FROWS

LOOKUPS = B * N_SPARSE
PER_W = LOOKUPS // NW
CHUNK = 128
NCH = PER_W // CHUNK


def _sc_repack(t2, tail):
    mesh = plsc.VectorSubcoreMesh(core_axis_name="c", subcore_axis_name="s")

    @functools.partial(
        pl.kernel,
        mesh=mesh,
        out_type=jax.ShapeDtypeStruct((OUTA, 128), jnp.float32),
        compiler_params=pltpu.CompilerParams(
            use_tc_tiling_on_sc=True, needs_layout_passes=False),
        scratch_types=[
            pltpu.VMEM((16, RWIN), jnp.float32),
            pltpu.VMEM((464, 128), jnp.float32),
            pltpu.VMEM((TAIL * K,), jnp.float32),
            pltpu.SemaphoreType.DMA,
        ],
    )
    def k(t2_hbm, tail_hbm, out_hbm, stage_v, outb_v, tail_v, sem):
        wid = lax.axis_index("s") * 2 + lax.axis_index("c")
        lanes = lax.iota(jnp.int32, 16)

        def transpose_window(nchunks):
            def vchunk(m, _):
                v0 = m * 16
                vv = v0 + lanes
                pos = lax.shift_right_logical(vv, 3) * 128 + \
                    jnp.bitwise_and(vv, 7) * 16
                for kk in range(16):
                    vals = stage_v[kk, pl.ds(v0, 16)]
                    pk = pos + kk
                    plsc.store_scatter(
                        outb_v, [lax.shift_right_logical(pk, 7),
                                 jnp.bitwise_and(pk, 127)], vals)
                return 0
            lax.fori_loop(0, nchunks, vchunk, 0)

        def task_body(q, _):
            tid = wid + NW * q

            @pl.when(tid < RTASK)
            def _():
                f = tid // RNW
                c = tid % RNW
                row0 = pl.multiple_of(f * 16, 8)
                r0 = f * FROWS + c * (RWIN // 8)

                @pl.when(c < RNW - 1)
                def _():
                    pltpu.sync_copy(
                        t2_hbm.at[pl.ds(row0, 16),
                                  pl.ds(pl.multiple_of(c * RWIN, 128), RWIN)],
                        stage_v)
                    transpose_window(RWIN // 16)
                    pltpu.sync_copy(
                        outb_v.at[pl.ds(0, RWIN // 8)],
                        out_hbm.at[pl.ds(r0, RWIN // 8)])

                @pl.when(c == RNW - 1)
                def _():
                    pltpu.sync_copy(
                        t2_hbm.at[pl.ds(row0, 16),
                                  pl.ds(pl.multiple_of(27 * RWIN, 128), RLAST)],
                        stage_v.at[:, pl.ds(0, RLAST)])
                    pltpu.sync_copy(tail_hbm.at[f], tail_v)
                    transpose_window(RLAST // 16)

                    def tchunk(m, _):
                        vv = m * 16 + lanes
                        pos = lax.shift_right_logical(RLAST + vv, 3) * 128 + \
                            jnp.bitwise_and(vv, 7) * 16
                        for kk in range(16):
                            tv = plsc.load_gather(tail_v, [vv * 16 + kk])
                            pk = pos + kk
                            plsc.store_scatter(
                                outb_v, [lax.shift_right_logical(pk, 7),
                                         jnp.bitwise_and(pk, 127)], tv)
                        return 0
                    lax.fori_loop(0, 2, tchunk, 0)
                    pltpu.sync_copy(
                        outb_v.at[pl.ds(0, RLAST // 8 + 8)],
                        out_hbm.at[pl.ds(r0, RLAST // 8 + 8)])

            return 0

        lax.fori_loop(0, RTPW, task_body, 0)

    return k(t2, tail)


def _sc_gather(table, idx3):
    mesh = plsc.VectorSubcoreMesh(core_axis_name="c", subcore_axis_name="s")

    @functools.partial(
        pl.kernel,
        mesh=mesh,
        out_type=jax.ShapeDtypeStruct((LOOKUPS, K), jnp.float32),
        compiler_params=pltpu.CompilerParams(use_tc_tiling_on_sc=False),
        scratch_types=[
            pltpu.VMEM((NCH, CHUNK), jnp.int32),
            pltpu.VMEM((PER_W, K), jnp.float32),
            pltpu.SemaphoreType.DMA,
        ],
    )
    def k(table_hbm, idx_hbm, out_hbm, idx_v, rows_v, sem):
        wid = lax.axis_index("s") * 2 + lax.axis_index("c")
        pltpu.sync_copy(idx_hbm.at[wid], idx_v)
        handles = [
            pltpu.async_copy(
                table_hbm.at[idx_v.at[j]],
                rows_v.at[pl.ds(j * CHUNK, CHUNK)],
                sem,
            )
            for j in range(NCH)
        ]
        for h in handles:
            h.wait()
        pltpu.sync_copy(rows_v, out_hbm.at[pl.ds(wid * PER_W, PER_W)])

    return k(table, idx3)


def _mlp_body(se_ref, dp_ref, w1, b1r, w2, b2r, wo, bor, out_ref):
    dn = (((1,), (1,)), ((), ()))
    h = jnp.concatenate([se_ref[...], dp_ref[...]], axis=1)
    o = lax.dot_general(h, w1[...], dn,
                        preferred_element_type=jnp.float32) + b1r[...]
    h = jnp.maximum(h + o, 0.0)
    o = lax.dot_general(h, w2[...], dn,
                        preferred_element_type=jnp.float32) + b2r[...]
    h = jnp.maximum(h + o, 0.0)
    hb = h.astype(jnp.bfloat16).astype(jnp.float32)
    wob = wo[...].astype(jnp.bfloat16).astype(jnp.float32)
    s = jnp.sum(hb * wob, axis=1, keepdims=True) + bor[...]
    out_ref[...] = 1.0 / (1.0 + jnp.exp(-s))


def _tc_mlp(se, dp, w1p, b1p, w2p, b2p, wop, bop):
    bb = 1024
    return pl.pallas_call(
        _mlp_body,
        grid=(B // bb,),
        in_specs=[
            pl.BlockSpec((bb, D_SPARSE), lambda i: (i, 0)),
            pl.BlockSpec((bb, DENSE_PAD), lambda i: (i, 0)),
            pl.BlockSpec((DP, DP), lambda i: (0, 0)),
            pl.BlockSpec((1, DP), lambda i: (0, 0)),
            pl.BlockSpec((DP, DP), lambda i: (0, 0)),
            pl.BlockSpec((1, DP), lambda i: (0, 0)),
            pl.BlockSpec((1, DP), lambda i: (0, 0)),
            pl.BlockSpec((1, 1), lambda i: (0, 0)),
        ],
        out_specs=pl.BlockSpec((bb, 1), lambda i: (i, 0)),
        out_shape=jax.ShapeDtypeStruct((B, 1), jnp.float32),
    )(se, dp, w1p, b1p, w2p, b2p, wop, bop)


_PERM = np.concatenate([np.arange(N_DENSE, D), np.arange(N_DENSE)])


def kernel(x, emb, W1, b1, W2, b2, Wout, bout):
    x_dense = x[:, :N_DENSE]
    x_sparse = x[:, N_DENSE:].astype(jnp.int32)
    offs = jnp.arange(N_SPARSE, dtype=jnp.int32) * VSTRIDE
    idx3 = (x_sparse + offs[None, :]).reshape(NW, NCH, CHUNK)

    t2 = emb.transpose(0, 2, 1).reshape(D_SPARSE, VOCAB)
    tail = emb[:, VMAIN:, :].reshape(N_SPARSE, TAIL * K)
    table = _sc_repack(t2, tail).reshape(N_SPARSE * VSTRIDE, K)

    se = _sc_gather(table, idx3).reshape(B, D_SPARSE)
    dp = jnp.pad(x_dense, ((0, 0), (0, DENSE_PAD - N_DENSE)))

    def pw(w):
        return jnp.pad(w[_PERM][:, _PERM], ((0, DP - D), (0, DP - D)))

    w1p, w2p = pw(W1), pw(W2)
    b1p = jnp.pad(b1[_PERM], (0, DP - D)).reshape(1, DP)
    b2p = jnp.pad(b2[_PERM], (0, DP - D)).reshape(1, DP)
    wop = jnp.pad(Wout[:, _PERM], ((0, 0), (0, DP - D)))
    bop = bout.reshape(1, 1)

    return _tc_mlp(se, dp, w1p, b1p, w2p, b2p, wop, bop)


# speedup vs baseline: 2.3195x; 1.0027x over previous
import functools

import jax
import jax.numpy as jnp
import numpy as np
from jax import lax
from jax.experimental import pallas as pl
from jax.experimental.pallas import tpu as pltpu
from jax.experimental.pallas import tpu_sc as plsc

N_DENSE = 13
N_SPARSE = 26
K = 16
VOCAB = 100000
B = 4096
D_SPARSE = N_SPARSE * K
D = D_SPARSE + N_DENSE
DP = 512
DENSE_PAD = DP - D_SPARSE

NW = 32

VMAIN = VOCAB - VOCAB % 128
TAIL = VOCAB - VMAIN
RWIN = 3584
RNW = 28
RLAST = VMAIN - 27 * RWIN
RTASK = N_SPARSE * RNW
RTPW = (RTASK + NW - 1) // NW
VSTRIDE = 100032
FROWS = VSTRIDE // 8
OUTA = N_SPARSE * FROWS

LOOKUPS = B * N_SPARSE
PER_W = LOOKUPS // NW
CHUNK = 128
NCH = PER_W // CHUNK


def _sc_repack(t2, tail):
    mesh = plsc.VectorSubcoreMesh(core_axis_name="c", subcore_axis_name="s")

    @functools.partial(
        pl.kernel,
        mesh=mesh,
        out_type=jax.ShapeDtypeStruct((OUTA, 128), jnp.float32),
        compiler_params=pltpu.CompilerParams(
            use_tc_tiling_on_sc=True, needs_layout_passes=False),
        scratch_types=[
            pltpu.VMEM((16, RWIN), jnp.float32),
            pltpu.VMEM((464, 128), jnp.float32),
            pltpu.VMEM((TAIL * K,), jnp.float32),
            pltpu.SemaphoreType.DMA,
        ],
    )
    def k(t2_hbm, tail_hbm, out_hbm, stage_v, outb_v, tail_v, sem):
        wid = lax.axis_index("s") * 2 + lax.axis_index("c")
        lanes = lax.iota(jnp.int32, 16)

        def transpose_window(nchunks):
            def vchunk(m, _):
                v0 = m * 16
                vv = v0 + lanes
                pos = lax.shift_right_logical(vv, 3) * 128 + \
                    jnp.bitwise_and(vv, 7) * 16
                for kk in range(16):
                    vals = stage_v[kk, pl.ds(v0, 16)]
                    pk = pos + kk
                    plsc.store_scatter(
                        outb_v, [lax.shift_right_logical(pk, 7),
                                 jnp.bitwise_and(pk, 127)], vals)
                return 0
            lax.fori_loop(0, nchunks, vchunk, 0, unroll=4)

        def task_body(q, _):
            tid = wid + NW * q

            @pl.when(tid < RTASK)
            def _():
                f = tid // RNW
                c = tid % RNW
                row0 = pl.multiple_of(f * 16, 8)
                r0 = f * FROWS + c * (RWIN // 8)

                @pl.when(c < RNW - 1)
                def _():
                    pltpu.sync_copy(
                        t2_hbm.at[pl.ds(row0, 16),
                                  pl.ds(pl.multiple_of(c * RWIN, 128), RWIN)],
                        stage_v)
                    transpose_window(RWIN // 16)
                    pltpu.sync_copy(
                        outb_v.at[pl.ds(0, RWIN // 8)],
                        out_hbm.at[pl.ds(r0, RWIN // 8)])

                @pl.when(c == RNW - 1)
                def _():
                    pltpu.sync_copy(
                        t2_hbm.at[pl.ds(row0, 16),
                                  pl.ds(pl.multiple_of(27 * RWIN, 128), RLAST)],
                        stage_v.at[:, pl.ds(0, RLAST)])
                    pltpu.sync_copy(tail_hbm.at[f], tail_v)
                    transpose_window(RLAST // 16)

                    def tchunk(m, _):
                        vv = m * 16 + lanes
                        pos = lax.shift_right_logical(RLAST + vv, 3) * 128 + \
                            jnp.bitwise_and(vv, 7) * 16
                        for kk in range(16):
                            tv = plsc.load_gather(tail_v, [vv * 16 + kk])
                            pk = pos + kk
                            plsc.store_scatter(
                                outb_v, [lax.shift_right_logical(pk, 7),
                                         jnp.bitwise_and(pk, 127)], tv)
                        return 0
                    lax.fori_loop(0, 2, tchunk, 0)
                    pltpu.sync_copy(
                        outb_v.at[pl.ds(0, RLAST // 8 + 8)],
                        out_hbm.at[pl.ds(r0, RLAST // 8 + 8)])

            return 0

        lax.fori_loop(0, RTPW, task_body, 0)

    return k(t2, tail)


def _sc_gather(table, idx3):
    mesh = plsc.VectorSubcoreMesh(core_axis_name="c", subcore_axis_name="s")

    @functools.partial(
        pl.kernel,
        mesh=mesh,
        out_type=jax.ShapeDtypeStruct((LOOKUPS, K), jnp.float32),
        compiler_params=pltpu.CompilerParams(use_tc_tiling_on_sc=False),
        scratch_types=[
            pltpu.VMEM((NCH, CHUNK), jnp.int32),
            pltpu.VMEM((PER_W, K), jnp.float32),
            pltpu.SemaphoreType.DMA,
        ],
    )
    def k(table_hbm, idx_hbm, out_hbm, idx_v, rows_v, sem):
        wid = lax.axis_index("s") * 2 + lax.axis_index("c")
        pltpu.sync_copy(idx_hbm.at[wid], idx_v)
        handles = [
            pltpu.async_copy(
                table_hbm.at[idx_v.at[j]],
                rows_v.at[pl.ds(j * CHUNK, CHUNK)],
                sem,
            )
            for j in range(NCH)
        ]
        for h in handles:
            h.wait()
        pltpu.sync_copy(rows_v, out_hbm.at[pl.ds(wid * PER_W, PER_W)])

    return k(table, idx3)


def _mlp_body(se_ref, dp_ref, w1, b1r, w2, b2r, wo, bor, out_ref):
    dn = (((1,), (1,)), ((), ()))
    h = jnp.concatenate([se_ref[...], dp_ref[...]], axis=1)
    o = lax.dot_general(h, w1[...], dn,
                        preferred_element_type=jnp.float32) + b1r[...]
    h = jnp.maximum(h + o, 0.0)
    o = lax.dot_general(h, w2[...], dn,
                        preferred_element_type=jnp.float32) + b2r[...]
    h = jnp.maximum(h + o, 0.0)
    hb = h.astype(jnp.bfloat16).astype(jnp.float32)
    wob = wo[...].astype(jnp.bfloat16).astype(jnp.float32)
    s = jnp.sum(hb * wob, axis=1, keepdims=True) + bor[...]
    out_ref[...] = 1.0 / (1.0 + jnp.exp(-s))


def _tc_mlp(se, dp, w1p, b1p, w2p, b2p, wop, bop):
    bb = 1024
    return pl.pallas_call(
        _mlp_body,
        grid=(B // bb,),
        in_specs=[
            pl.BlockSpec((bb, D_SPARSE), lambda i: (i, 0)),
            pl.BlockSpec((bb, DENSE_PAD), lambda i: (i, 0)),
            pl.BlockSpec((DP, DP), lambda i: (0, 0)),
            pl.BlockSpec((1, DP), lambda i: (0, 0)),
            pl.BlockSpec((DP, DP), lambda i: (0, 0)),
            pl.BlockSpec((1, DP), lambda i: (0, 0)),
            pl.BlockSpec((1, DP), lambda i: (0, 0)),
            pl.BlockSpec((1, 1), lambda i: (0, 0)),
        ],
        out_specs=pl.BlockSpec((bb, 1), lambda i: (i, 0)),
        out_shape=jax.ShapeDtypeStruct((B, 1), jnp.float32),
    )(se, dp, w1p, b1p, w2p, b2p, wop, bop)


_PERM = np.concatenate([np.arange(N_DENSE, D), np.arange(N_DENSE)])


def kernel(x, emb, W1, b1, W2, b2, Wout, bout):
    x_dense = x[:, :N_DENSE]
    x_sparse = x[:, N_DENSE:].astype(jnp.int32)
    offs = jnp.arange(N_SPARSE, dtype=jnp.int32) * VSTRIDE
    idx3 = (x_sparse + offs[None, :]).reshape(NW, NCH, CHUNK)

    t2 = emb.transpose(0, 2, 1).reshape(D_SPARSE, VOCAB)
    tail = emb[:, VMAIN:, :].reshape(N_SPARSE, TAIL * K)
    table = _sc_repack(t2, tail).reshape(N_SPARSE * VSTRIDE, K)

    se = _sc_gather(table, idx3).reshape(B, D_SPARSE)
    dp = jnp.pad(x_dense, ((0, 0), (0, DENSE_PAD - N_DENSE)))

    def pw(w):
        return jnp.pad(w[_PERM][:, _PERM], ((0, DP - D), (0, DP - D)))

    w1p, w2p = pw(W1), pw(W2)
    b1p = jnp.pad(b1[_PERM], (0, DP - D)).reshape(1, DP)
    b2p = jnp.pad(b2[_PERM], (0, DP - D)).reshape(1, DP)
    wop = jnp.pad(Wout[:, _PERM], ((0, 0), (0, DP - D)))
    bop = bout.reshape(1, 1)

    return _tc_mlp(se, dp, w1p, b1p, w2p, b2p, wop, bop)


# speedup vs baseline: 2.9833x; 1.2862x over previous
import functools

import jax
import jax.numpy as jnp
import numpy as np
from jax import lax
from jax.experimental import pallas as pl
from jax.experimental.pallas import tpu as pltpu
from jax.experimental.pallas import tpu_sc as plsc

N_DENSE = 13
N_SPARSE = 26
K = 16
VOCAB = 100000
B = 4096
D_SPARSE = N_SPARSE * K
D = D_SPARSE + N_DENSE
DP = 512
DENSE_PAD = DP - D_SPARSE

NW = 32

VMAIN = VOCAB - VOCAB % 128
TAIL = VOCAB - VMAIN
RWIN = 1792
RNW = 56
RLASTOFF = VMAIN - RWIN
RVBASE = 55 * RWIN - RLASTOFF
RTASK = N_SPARSE * RNW
RTPW = (RTASK + NW - 1) // NW
ORPT = RWIN // 8
VSTRIDE = 100352
FROWS = VSTRIDE // 8
OUTA = N_SPARSE * FROWS

LOOKUPS = B * N_SPARSE
PER_W = LOOKUPS // NW
CHUNK = 128
NCH = PER_W // CHUNK


def _sc_repack(t2, tail):
    mesh = plsc.VectorSubcoreMesh(core_axis_name="c", subcore_axis_name="s")

    @functools.partial(
        pl.kernel,
        mesh=mesh,
        out_type=jax.ShapeDtypeStruct((OUTA, 128), jnp.float32),
        compiler_params=pltpu.CompilerParams(
            use_tc_tiling_on_sc=True, needs_layout_passes=False),
        scratch_types=[
            pltpu.VMEM((16, RWIN), jnp.float32),
            pltpu.VMEM((16, RWIN), jnp.float32),
            pltpu.VMEM((ORPT, 128), jnp.float32),
            pltpu.VMEM((ORPT, 128), jnp.float32),
            pltpu.VMEM((TAIL * K,), jnp.float32),
            pltpu.VMEM((TAIL * K,), jnp.float32),
            pltpu.SemaphoreType.DMA,
            pltpu.SemaphoreType.DMA,
            pltpu.SemaphoreType.DMA,
            pltpu.SemaphoreType.DMA,
            pltpu.SemaphoreType.DMA,
            pltpu.SemaphoreType.DMA,
        ],
    )
    def k(t2_hbm, tail_hbm, out_hbm, st0, st1, ob0, ob1, tl0, tl1,
          ss0, ss1, ts0, ts1, ws0, ws1):
        wid = lax.axis_index("s") * 2 + lax.axis_index("c")
        lanes = lax.iota(jnp.int32, 16)
        stage = [st0, st1]
        outb = [ob0, ob1]
        tailb = [tl0, tl1]
        ssem = [ss0, ss1]
        tsem = [ts0, ts1]
        wsem = [ws0, ws1]

        def decode(q):
            tid = jnp.minimum(wid + NW * q, RTASK - 1)
            f = tid // RNW
            c = tid % RNW
            return f, c

        def start_stage(q):
            f, c = decode(q)
            row0 = pl.multiple_of(f * 16, 8)
            off = pl.multiple_of(
                jnp.where(c == RNW - 1, RLASTOFF, c * RWIN), 128)
            p = q % 2
            hs = pltpu.async_copy(
                t2_hbm.at[pl.ds(row0, 16), pl.ds(off, RWIN)],
                stage[p], ssem[p])
            ht = pltpu.async_copy(tail_hbm.at[f], tailb[p], tsem[p])
            return hs, ht

        def compute(q):
            f, c = decode(q)
            p = q % 2
            last = c == RNW - 1
            vbase = jnp.where(last, RVBASE, 0)
            nch = jnp.where(last, (RWIN - RVBASE) // 16, RWIN // 16)

            def vchunk(m, _):
                x = m * 16 + lanes
                pos = lax.shift_right_logical(x, 3) * 128 + \
                    jnp.bitwise_and(x, 7) * 16
                for kk in range(16):
                    vals = stage[p][kk, pl.ds(vbase + m * 16, 16)]
                    pk = pos + kk
                    plsc.store_scatter(
                        outb[p], [lax.shift_right_logical(pk, 7),
                                  jnp.bitwise_and(pk, 127)], vals)
                return 0
            lax.fori_loop(0, nch, vchunk, 0)

            @pl.when(last)
            def _():
                def titer(i, _):
                    m = lax.shift_right_logical(i, 4)
                    kk = jnp.bitwise_and(i, 15)
                    x = (RWIN - RVBASE) + m * 16 + lanes
                    xr = m * 16 + lanes
                    tv = plsc.load_gather(tailb[p], [xr * 16 + kk])
                    pk = lax.shift_right_logical(x, 3) * 128 + \
                        jnp.bitwise_and(x, 7) * 16 + kk
                    plsc.store_scatter(
                        outb[p], [lax.shift_right_logical(pk, 7),
                                  jnp.bitwise_and(pk, 127)], tv)
                    return 0
                lax.fori_loop(0, 2 * 16, titer, 0)

        def start_write(q):
            f, c = decode(q)
            p = q % 2
            r0 = f * FROWS + c * ORPT
            return pltpu.async_copy(
                outb[p], out_hbm.at[pl.ds(r0, ORPT)], wsem[p])

        hs, ht = start_stage(0)
        hw = [None, None]
        for q in range(RTPW):
            hs.wait()
            ht.wait()
            if q + 1 < RTPW:
                hs, ht = start_stage(q + 1)
            if hw[q % 2] is not None:
                hw[q % 2].wait()
            compute(q)
            hw[q % 2] = start_write(q)
        hw[0].wait()
        hw[1].wait()

    return k(t2, tail)


def _sc_gather(table, idx3):
    mesh = plsc.VectorSubcoreMesh(core_axis_name="c", subcore_axis_name="s")

    @functools.partial(
        pl.kernel,
        mesh=mesh,
        out_type=jax.ShapeDtypeStruct((LOOKUPS, K), jnp.float32),
        compiler_params=pltpu.CompilerParams(use_tc_tiling_on_sc=False),
        scratch_types=[
            pltpu.VMEM((NCH, CHUNK), jnp.int32),
            pltpu.VMEM((PER_W, K), jnp.float32),
            pltpu.SemaphoreType.DMA,
        ],
    )
    def k(table_hbm, idx_hbm, out_hbm, idx_v, rows_v, sem):
        wid = lax.axis_index("s") * 2 + lax.axis_index("c")
        pltpu.sync_copy(idx_hbm.at[wid], idx_v)
        handles = [
            pltpu.async_copy(
                table_hbm.at[idx_v.at[j]],
                rows_v.at[pl.ds(j * CHUNK, CHUNK)],
                sem,
            )
            for j in range(NCH)
        ]
        for h in handles:
            h.wait()
        pltpu.sync_copy(rows_v, out_hbm.at[pl.ds(wid * PER_W, PER_W)])

    return k(table, idx3)


def _mlp_body(se_ref, dp_ref, w1, b1r, w2, b2r, wo, bor, out_ref):
    dn = (((1,), (1,)), ((), ()))
    h = jnp.concatenate([se_ref[...], dp_ref[...]], axis=1)
    o = lax.dot_general(h, w1[...], dn,
                        preferred_element_type=jnp.float32) + b1r[...]
    h = jnp.maximum(h + o, 0.0)
    o = lax.dot_general(h, w2[...], dn,
                        preferred_element_type=jnp.float32) + b2r[...]
    h = jnp.maximum(h + o, 0.0)
    hb = h.astype(jnp.bfloat16).astype(jnp.float32)
    wob = wo[...].astype(jnp.bfloat16).astype(jnp.float32)
    s = jnp.sum(hb * wob, axis=1, keepdims=True) + bor[...]
    out_ref[...] = 1.0 / (1.0 + jnp.exp(-s))


def _tc_mlp(se, dp, w1p, b1p, w2p, b2p, wop, bop):
    bb = 1024
    return pl.pallas_call(
        _mlp_body,
        grid=(B // bb,),
        in_specs=[
            pl.BlockSpec((bb, D_SPARSE), lambda i: (i, 0)),
            pl.BlockSpec((bb, DENSE_PAD), lambda i: (i, 0)),
            pl.BlockSpec((DP, DP), lambda i: (0, 0)),
            pl.BlockSpec((1, DP), lambda i: (0, 0)),
            pl.BlockSpec((DP, DP), lambda i: (0, 0)),
            pl.BlockSpec((1, DP), lambda i: (0, 0)),
            pl.BlockSpec((1, DP), lambda i: (0, 0)),
            pl.BlockSpec((1, 1), lambda i: (0, 0)),
        ],
        out_specs=pl.BlockSpec((bb, 1), lambda i: (i, 0)),
        out_shape=jax.ShapeDtypeStruct((B, 1), jnp.float32),
    )(se, dp, w1p, b1p, w2p, b2p, wop, bop)


_PERM = np.concatenate([np.arange(N_DENSE, D), np.arange(N_DENSE)])


def kernel(x, emb, W1, b1, W2, b2, Wout, bout):
    x_dense = x[:, :N_DENSE]
    x_sparse = x[:, N_DENSE:].astype(jnp.int32)
    offs = jnp.arange(N_SPARSE, dtype=jnp.int32) * VSTRIDE
    idx3 = (x_sparse + offs[None, :]).reshape(NW, NCH, CHUNK)

    t2 = emb.transpose(0, 2, 1).reshape(D_SPARSE, VOCAB)
    tail = emb[:, VMAIN:, :].reshape(N_SPARSE, TAIL * K)
    table = _sc_repack(t2, tail).reshape(N_SPARSE * VSTRIDE, K)

    se = _sc_gather(table, idx3).reshape(B, D_SPARSE)
    dp = jnp.pad(x_dense, ((0, 0), (0, DENSE_PAD - N_DENSE)))

    def pw(w):
        return jnp.pad(w[_PERM][:, _PERM], ((0, DP - D), (0, DP - D)))

    w1p, w2p = pw(W1), pw(W2)
    b1p = jnp.pad(b1[_PERM], (0, DP - D)).reshape(1, DP)
    b2p = jnp.pad(b2[_PERM], (0, DP - D)).reshape(1, DP)
    wop = jnp.pad(Wout[:, _PERM], ((0, 0), (0, DP - D)))
    bop = bout.reshape(1, 1)

    return _tc_mlp(se, dp, w1p, b1p, w2p, b2p, wop, bop)
